# Initial kernel scaffold; baseline (speedup 1.0000x reference)
import jax
import jax.numpy as jnp
from jax.experimental import pallas as pl


def kernel(x, static, time, sensor_mask, params):
    raise NotImplementedError("write your pallas kernel here")


# speedup vs baseline: 47.3973x; 47.3973x over previous
import math

import jax
import jax.numpy as jnp
from jax.experimental import pallas as pl
from jax.experimental.pallas import tpu as pltpu

B = 8
M = 36
T = 512
STATIC = 16
POS = 128
PHI_IN = M + POS + 1
LAT = 128
DOT = 64
H = 4
OUT = 2
MAX_TS = 1000.0

TT = 128
NT = T // TT


def _phi_tail(h1, pW2, pb2, pW3, pb3, pW4, pb4):
    h2 = jax.nn.relu(jnp.dot(h1, pW2, preferred_element_type=jnp.float32) + pb2)
    h3 = jax.nn.relu(jnp.dot(h2, pW3, preferred_element_type=jnp.float32) + pb3)
    return jnp.dot(h3, pW4, preferred_element_type=jnp.float32) + pb4


def _fused_kernel(
    x_ref, time_ref, sm_ref, static_ref,
    dW1_ref, db1_ref, dW2_ref, db2_ref,
    pW1_ref, pb1_ref, pW2_ref, pb2_ref, pW3_ref, pb3_ref, pW4_ref, pb4_ref,
    Wk_ref, Wq_ref,
    rW1_ref, rb1_ref, rW2_ref, rb2_ref, rW3_ref, rb3_ref,
    out_ref,
    m_ref, d_ref, num_ref,
):
    nt = pl.program_id(1)

    cols = []
    for h in range(H):
        wq_h = jnp.transpose(Wq_ref[h:h + 1, :])
        cols.append(jnp.dot(Wk_ref[0:PHI_IN, h * DOT:(h + 1) * DOT], wq_h,
                            preferred_element_type=jnp.float32))
    Wkq = jnp.concatenate(cols, axis=1) * (1.0 / math.sqrt(DOT))

    pW1 = pW1_ref[...]
    pb1 = pb1_ref[...]
    pW2 = pW2_ref[...]
    pb2 = pb2_ref[...]
    pW3 = pW3_ref[...]
    pb3 = pb3_ref[...]
    pW4 = pW4_ref[...]
    pb4 = pb4_ref[...]

    tcol = jnp.transpose(time_ref[0])
    k = jax.lax.broadcasted_iota(jnp.int32, (1, POS // 2), 1).astype(jnp.float32)
    inv_ts = jnp.exp(k * (-math.log(MAX_TS) / (POS // 2 - 1)))
    scaled = tcol * inv_ts
    pe = jnp.concatenate([jnp.sin(scaled), jnp.cos(scaled)], axis=1)

    P1 = jnp.dot(pe, pW1[0:POS, :], preferred_element_type=jnp.float32)
    w1val = pW1[POS:POS + 1, :]
    W1m = pW1[POS + 1:PHI_IN, :]

    xv = x_ref[0]
    h1 = jax.nn.relu(
        P1[None, :, :]
        + xv[:, :, None] * w1val[None, :, :]
        + W1m[:, None, :]
        + pb1[None, :, :]
    )
    enc = _phi_tail(h1.reshape(M * TT, LAT), pW2, pb2, pW3, pb3, pW4, pb4)

    Pq = jnp.dot(pe, Wkq[0:POS, :], preferred_element_type=jnp.float32)
    base3 = (
        Pq[None, :, :]
        + xv[:, :, None] * Wkq[POS:POS + 1, :][None, :, :]
        + Wkq[POS + 1:PHI_IN, :][:, None, :]
    )
    validf3 = (sm_ref[0] != 0).astype(jnp.float32)[:, :, None]
    masked3 = base3 + (validf3 - 1.0) * 1e30
    masked2 = masked3.reshape(M * TT, H)

    @pl.when(nt == 0)
    def _init():
        s = static_ref[0]
        dh = jax.nn.relu(jnp.dot(s, dW1_ref[...],
                                 preferred_element_type=jnp.float32) + db1_ref[...])
        demo_enc = jnp.dot(dh, dW2_ref[...],
                           preferred_element_type=jnp.float32) + db2_ref[...]
        e1 = jax.nn.relu(jnp.dot(demo_enc, pW1,
                                 preferred_element_type=jnp.float32) + pb1)
        enc_d = _phi_tail(e1, pW2, pb2, pW3, pb3, pW4, pb4)
        base_d = jnp.dot(demo_enc, Wkq, preferred_element_type=jnp.float32)
        m_ref[...] = base_d
        d_ref[...] = jnp.ones((1, H), jnp.float32)
        num_ref[...] = jnp.broadcast_to(enc_d, (H, LAT))

    mt = jnp.max(masked2, axis=0, keepdims=True)
    m_old = m_ref[...]
    m_new = jnp.maximum(m_old, mt)
    scale = jnp.exp(m_old - m_new)
    ex = jnp.exp(base3 - m_new[None, :, :]) * validf3
    ex2 = ex.reshape(M * TT, H)
    d_ref[...] = d_ref[...] * scale + jnp.sum(ex2, axis=0, keepdims=True)
    num_ref[...] = num_ref[...] * jnp.transpose(scale) + jax.lax.dot_general(
        ex2, enc, (((0,), (0,)), ((), ())),
        preferred_element_type=jnp.float32)
    m_ref[...] = m_new

    @pl.when(nt == NT - 1)
    def _fini():
        pooled = num_ref[...] / jnp.transpose(d_ref[...])
        z = rb1_ref[...]
        for h in range(H):
            z = z + jnp.dot(pooled[h:h + 1, :],
                            rW1_ref[h * LAT:(h + 1) * LAT, :],
                            preferred_element_type=jnp.float32)
        z = jax.nn.relu(z)
        z = jax.nn.relu(jnp.dot(z, rW2_ref[...],
                                preferred_element_type=jnp.float32) + rb2_ref[...])
        out_ref[0] = jnp.dot(z, rW3_ref[...],
                             preferred_element_type=jnp.float32) + rb3_ref[...]


def kernel(x, static, time, sensor_mask, params):
    dW, db = params["demo"]
    pW, pb = params["phi"]
    rW, rb = params["rho"]
    Wk = params["W_k"]
    Wq = params["W_q"]

    row = lambda v: v.reshape(1, -1)

    full = lambda a: pl.BlockSpec(a.shape, lambda b, nt: (0,) * a.ndim)
    weights = [dW[0], row(db[0]), dW[1], row(db[1]),
               pW[0], row(pb[0]), pW[1], row(pb[1]),
               pW[2], row(pb[2]), pW[3], row(pb[3]),
               Wk, Wq,
               rW[0], row(rb[0]), rW[1], row(rb[1]), rW[2], row(rb[2])]

    out = pl.pallas_call(
        _fused_kernel,
        grid=(B, NT),
        in_specs=[
            pl.BlockSpec((1, M, TT), lambda b, nt: (b, 0, nt)),
            pl.BlockSpec((1, 1, TT), lambda b, nt: (b, 0, nt)),
            pl.BlockSpec((1, M, TT), lambda b, nt: (b, 0, nt)),
            pl.BlockSpec((1, 1, STATIC), lambda b, nt: (b, 0, 0)),
        ] + [full(w) for w in weights],
        out_specs=pl.BlockSpec((1, 1, OUT), lambda b, nt: (b, 0, 0)),
        out_shape=jax.ShapeDtypeStruct((B, 1, OUT), jnp.float32),
        scratch_shapes=[
            pltpu.VMEM((1, H), jnp.float32),
            pltpu.VMEM((1, H), jnp.float32),
            pltpu.VMEM((H, LAT), jnp.float32),
        ],
        compiler_params=pltpu.CompilerParams(
            dimension_semantics=("arbitrary", "arbitrary")),
    )(x, time.reshape(B, 1, T), sensor_mask, static.reshape(B, 1, STATIC),
      *weights)
    return out.reshape(B, OUT)


# speedup vs baseline: 59.8468x; 1.2627x over previous
import math

import jax
import jax.numpy as jnp
from jax.experimental import pallas as pl
from jax.experimental.pallas import tpu as pltpu

B = 8
M = 36
T = 512
STATIC = 16
POS = 128
PHI_IN = M + POS + 1
LAT = 128
DOT = 64
H = 4
OUT = 2
MAX_TS = 1000.0

TT = 128
NT = T // TT


def _phi_mid(h1, pW2, pb2, pW3, pb3):
    h2 = jax.nn.relu(jnp.dot(h1, pW2, preferred_element_type=jnp.float32) + pb2)
    return jax.nn.relu(jnp.dot(h2, pW3, preferred_element_type=jnp.float32) + pb3)


def _fused_kernel(
    x_ref, time_ref, sm_ref, static_ref,
    dW1_ref, db1_ref, dW2_ref, db2_ref,
    pW1_ref, pb1_ref, pW2_ref, pb2_ref, pW3_ref, pb3_ref, pW4_ref, pb4_ref,
    Wk_ref, Wq_ref,
    rW1_ref, rb1_ref, rW2_ref, rb2_ref, rW3_ref, rb3_ref,
    out_ref,
    m_ref, d_ref, num_ref, wkq_ref,
):
    nt = pl.program_id(1)

    @pl.when(jnp.logical_and(pl.program_id(0) == 0, nt == 0))
    def _fold():
        cols = []
        for h in range(H):
            wq_h = jnp.transpose(Wq_ref[h:h + 1, :])
            cols.append(jnp.dot(Wk_ref[0:PHI_IN, h * DOT:(h + 1) * DOT], wq_h,
                                preferred_element_type=jnp.float32))
        wkq_ref[...] = jnp.concatenate(cols, axis=1) * (1.0 / math.sqrt(DOT))

    Wkq = wkq_ref[...]

    pW1 = pW1_ref[...]
    pb1 = pb1_ref[...]
    pW2 = pW2_ref[...]
    pb2 = pb2_ref[...]
    pW3 = pW3_ref[...]
    pb3 = pb3_ref[...]
    pW4 = pW4_ref[...]
    pb4 = pb4_ref[...]

    tcol = jnp.transpose(time_ref[0])
    k = jax.lax.broadcasted_iota(jnp.int32, (1, POS // 2), 1).astype(jnp.float32)
    inv_ts = jnp.exp(k * (-math.log(MAX_TS) / (POS // 2 - 1)))
    scaled = tcol * inv_ts
    pe = jnp.concatenate([jnp.sin(scaled), jnp.cos(scaled)], axis=1)

    P1 = jnp.dot(pe, pW1[0:POS, :], preferred_element_type=jnp.float32)
    w1val = pW1[POS:POS + 1, :]
    W1m = pW1[POS + 1:PHI_IN, :]

    xv = x_ref[0]
    P1b = P1 + pb1
    h1 = jax.nn.relu(
        P1b[None, :, :]
        + xv[:, :, None] * w1val[None, :, :]
        + W1m[:, None, :]
    )
    h3 = _phi_mid(h1.reshape(M * TT, LAT), pW2, pb2, pW3, pb3)

    xmask = jnp.where(sm_ref[0] != 0, 0.0, -1e30)
    Pq = jnp.dot(pe, Wkq[0:POS, :], preferred_element_type=jnp.float32)
    masked3 = (
        Pq[None, :, :]
        + xv[:, :, None] * Wkq[POS:POS + 1, :][None, :, :]
        + Wkq[POS + 1:PHI_IN, :][:, None, :]
        + xmask[:, :, None]
    )
    masked2 = masked3.reshape(M * TT, H)

    @pl.when(nt == 0)
    def _init():
        s = static_ref[0]
        dh = jax.nn.relu(jnp.dot(s, dW1_ref[...],
                                 preferred_element_type=jnp.float32) + db1_ref[...])
        demo_enc = jnp.dot(dh, dW2_ref[...],
                           preferred_element_type=jnp.float32) + db2_ref[...]
        e1 = jax.nn.relu(jnp.dot(demo_enc, pW1,
                                 preferred_element_type=jnp.float32) + pb1)
        h3_d = _phi_mid(e1, pW2, pb2, pW3, pb3)
        base_d = jnp.dot(demo_enc, Wkq, preferred_element_type=jnp.float32)
        m_ref[...] = base_d
        d_ref[...] = jnp.ones((1, H), jnp.float32)
        num_ref[...] = jnp.broadcast_to(h3_d, (H, LAT))

    mt = jnp.max(masked2, axis=0, keepdims=True)
    m_old = m_ref[...]
    m_new = jnp.maximum(m_old, mt)
    scale = jnp.exp(m_old - m_new)
    ex2 = jnp.exp(masked2 - m_new)
    d_ref[...] = d_ref[...] * scale + jnp.sum(ex2, axis=0, keepdims=True)
    num_ref[...] = num_ref[...] * jnp.transpose(scale) + jax.lax.dot_general(
        ex2, h3, (((0,), (0,)), ((), ())),
        preferred_element_type=jnp.float32)
    m_ref[...] = m_new

    @pl.when(nt == NT - 1)
    def _fini():
        d_col = jnp.transpose(d_ref[...])
        pooled = (jnp.dot(num_ref[...], pW4,
                          preferred_element_type=jnp.float32)
                  + d_col * pb4) / d_col
        z = rb1_ref[...]
        for h in range(H):
            z = z + jnp.dot(pooled[h:h + 1, :],
                            rW1_ref[h * LAT:(h + 1) * LAT, :],
                            preferred_element_type=jnp.float32)
        z = jax.nn.relu(z)
        z = jax.nn.relu(jnp.dot(z, rW2_ref[...],
                                preferred_element_type=jnp.float32) + rb2_ref[...])
        out_ref[0] = jnp.dot(z, rW3_ref[...],
                             preferred_element_type=jnp.float32) + rb3_ref[...]


def kernel(x, static, time, sensor_mask, params):
    dW, db = params["demo"]
    pW, pb = params["phi"]
    rW, rb = params["rho"]
    Wk = params["W_k"]
    Wq = params["W_q"]

    row = lambda v: v.reshape(1, -1)

    full = lambda a: pl.BlockSpec(a.shape, lambda b, nt: (0,) * a.ndim)
    weights = [dW[0], row(db[0]), dW[1], row(db[1]),
               pW[0], row(pb[0]), pW[1], row(pb[1]),
               pW[2], row(pb[2]), pW[3], row(pb[3]),
               Wk, Wq,
               rW[0], row(rb[0]), rW[1], row(rb[1]), rW[2], row(rb[2])]

    out = pl.pallas_call(
        _fused_kernel,
        grid=(B, NT),
        in_specs=[
            pl.BlockSpec((1, M, TT), lambda b, nt: (b, 0, nt)),
            pl.BlockSpec((1, 1, TT), lambda b, nt: (b, 0, nt)),
            pl.BlockSpec((1, M, TT), lambda b, nt: (b, 0, nt)),
            pl.BlockSpec((1, 1, STATIC), lambda b, nt: (b, 0, 0)),
        ] + [full(w) for w in weights],
        out_specs=pl.BlockSpec((1, 1, OUT), lambda b, nt: (b, 0, 0)),
        out_shape=jax.ShapeDtypeStruct((B, 1, OUT), jnp.float32),
        scratch_shapes=[
            pltpu.VMEM((1, H), jnp.float32),
            pltpu.VMEM((1, H), jnp.float32),
            pltpu.VMEM((H, LAT), jnp.float32),
            pltpu.VMEM((PHI_IN, H), jnp.float32),
        ],
        compiler_params=pltpu.CompilerParams(
            dimension_semantics=("arbitrary", "arbitrary")),
    )(x, time.reshape(B, 1, T), sensor_mask, static.reshape(B, 1, STATIC),
      *weights)
    return out.reshape(B, OUT)


# speedup vs baseline: 72.0151x; 1.2033x over previous
import math

import jax
import jax.numpy as jnp
from jax.experimental import pallas as pl
from jax.experimental.pallas import tpu as pltpu

B = 8
M = 36
T = 512
STATIC = 16
POS = 128
PHI_IN = M + POS + 1
LAT = 128
DOT = 64
H = 4
OUT = 2
MAX_TS = 1000.0


def _phi_mid(h1, pW2, pb2, pW3, pb3):
    h2 = jax.nn.relu(jnp.dot(h1, pW2, preferred_element_type=jnp.float32) + pb2)
    return jax.nn.relu(jnp.dot(h2, pW3, preferred_element_type=jnp.float32) + pb3)


def _fused_kernel(
    x_ref, time_ref, sm_ref, static_ref,
    dW1_ref, db1_ref, dW2_ref, db2_ref,
    pW1_ref, pb1_ref, pW2_ref, pb2_ref, pW3_ref, pb3_ref, pW4_ref, pb4_ref,
    Wk_ref, Wq_ref,
    rW1_ref, rb1_ref, rW2_ref, rb2_ref, rW3_ref, rb3_ref,
    out_ref,
    wkq_ref,
):
    @pl.when(pl.program_id(0) == 0)
    def _fold():
        cols = []
        for h in range(H):
            wq_h = jnp.transpose(Wq_ref[h:h + 1, :])
            cols.append(jnp.dot(Wk_ref[0:PHI_IN, h * DOT:(h + 1) * DOT], wq_h,
                                preferred_element_type=jnp.float32))
        wkq_ref[...] = jnp.concatenate(cols, axis=1) * (1.0 / math.sqrt(DOT))

    Wkq = wkq_ref[...]

    pW1 = pW1_ref[...]
    pb1 = pb1_ref[...]
    pW2 = pW2_ref[...]
    pb2 = pb2_ref[...]
    pW3 = pW3_ref[...]
    pb3 = pb3_ref[...]

    tcol = jnp.transpose(time_ref[0])
    k = jax.lax.broadcasted_iota(jnp.int32, (1, POS // 2), 1).astype(jnp.float32)
    inv_ts = jnp.exp(k * (-math.log(MAX_TS) / (POS // 2 - 1)))
    scaled = tcol * inv_ts
    pe = jnp.concatenate([jnp.sin(scaled), jnp.cos(scaled)], axis=1)

    P1b = jnp.dot(pe, pW1[0:POS, :],
                  preferred_element_type=jnp.float32) + pb1
    w1val = pW1[POS:POS + 1, :]
    W1m = pW1[POS + 1:PHI_IN, :]

    xv = x_ref[0]
    h1 = jax.nn.relu(
        P1b[None, :, :]
        + xv[:, :, None] * w1val[None, :, :]
        + W1m[:, None, :]
    )
    h3 = _phi_mid(h1.reshape(M * T, LAT), pW2, pb2, pW3, pb3)

    xmask = jnp.where(sm_ref[0] != 0, 0.0, -1e30)
    Pq = jnp.dot(pe, Wkq[0:POS, :], preferred_element_type=jnp.float32)
    masked3 = (
        Pq[None, :, :]
        + xv[:, :, None] * Wkq[POS:POS + 1, :][None, :, :]
        + Wkq[POS + 1:PHI_IN, :][:, None, :]
        + xmask[:, :, None]
    )
    masked2 = masked3.reshape(M * T, H)

    s = static_ref[0]
    dh = jax.nn.relu(jnp.dot(s, dW1_ref[...],
                             preferred_element_type=jnp.float32) + db1_ref[...])
    demo_enc = jnp.dot(dh, dW2_ref[...],
                       preferred_element_type=jnp.float32) + db2_ref[...]
    e1 = jax.nn.relu(jnp.dot(demo_enc, pW1,
                             preferred_element_type=jnp.float32) + pb1)
    h3_d = _phi_mid(e1, pW2, pb2, pW3, pb3)
    base_d = jnp.dot(demo_enc, Wkq, preferred_element_type=jnp.float32)

    mt = jnp.max(masked2, axis=0, keepdims=True)
    m = jnp.maximum(base_d, mt)
    ex2 = jnp.exp(masked2 - m)
    e_d = jnp.exp(base_d - m)
    den = e_d + jnp.sum(ex2, axis=0, keepdims=True)
    num = jax.lax.dot_general(
        ex2, h3, (((0,), (0,)), ((), ())),
        preferred_element_type=jnp.float32) + jnp.transpose(e_d) * h3_d

    d_col = jnp.transpose(den)
    pooled = (jnp.dot(num, pW4_ref[...],
                      preferred_element_type=jnp.float32)
              + d_col * pb4_ref[...]) / d_col
    z = rb1_ref[...]
    for h in range(H):
        z = z + jnp.dot(pooled[h:h + 1, :],
                        rW1_ref[h * LAT:(h + 1) * LAT, :],
                        preferred_element_type=jnp.float32)
    z = jax.nn.relu(z)
    z = jax.nn.relu(jnp.dot(z, rW2_ref[...],
                            preferred_element_type=jnp.float32) + rb2_ref[...])
    out_ref[0] = jnp.dot(z, rW3_ref[...],
                         preferred_element_type=jnp.float32) + rb3_ref[...]


def kernel(x, static, time, sensor_mask, params):
    dW, db = params["demo"]
    pW, pb = params["phi"]
    rW, rb = params["rho"]
    Wk = params["W_k"]
    Wq = params["W_q"]

    row = lambda v: v.reshape(1, -1)

    full = lambda a: pl.BlockSpec(a.shape, lambda b: (0,) * a.ndim)
    weights = [dW[0], row(db[0]), dW[1], row(db[1]),
               pW[0], row(pb[0]), pW[1], row(pb[1]),
               pW[2], row(pb[2]), pW[3], row(pb[3]),
               Wk, Wq,
               rW[0], row(rb[0]), rW[1], row(rb[1]), rW[2], row(rb[2])]

    out = pl.pallas_call(
        _fused_kernel,
        grid=(B,),
        in_specs=[
            pl.BlockSpec((1, M, T), lambda b: (b, 0, 0)),
            pl.BlockSpec((1, 1, T), lambda b: (b, 0, 0)),
            pl.BlockSpec((1, M, T), lambda b: (b, 0, 0)),
            pl.BlockSpec((1, 1, STATIC), lambda b: (b, 0, 0)),
        ] + [full(w) for w in weights],
        out_specs=pl.BlockSpec((1, 1, OUT), lambda b: (b, 0, 0)),
        out_shape=jax.ShapeDtypeStruct((B, 1, OUT), jnp.float32),
        scratch_shapes=[
            pltpu.VMEM((PHI_IN, H), jnp.float32),
        ],
        compiler_params=pltpu.CompilerParams(
            dimension_semantics=("arbitrary",)),
    )(x, time.reshape(B, 1, T), sensor_mask, static.reshape(B, 1, STATIC),
      *weights)
    return out.reshape(B, OUT)
